# baseline (speedup 1.0000x reference)
import jax
import jax.numpy as jnp
from jax import lax
from jax.experimental import pallas as pl

D = 64
S = 64
BLKB2 = 2048


def _tc_body(tbl_ref, tbl21_ref, p_ref, c_ref, out_ref):
    def valu_square(si):
        tbl = tbl_ref[si]

        def col(k):
            return tbl[:, k][:, None]

        p = p_ref[si][None, :]
        c = c_ref[si][None, :]
        b0 = (c & 1) != 0
        b1 = (c & 2) != 0
        b2 = (p & 1) != 0
        b3 = (p & 2) != 0
        b4 = (p & 4) != 0
        t0 = jnp.where(b2, col(1), col(0))
        t1 = jnp.where(b2, col(3), col(2))
        t2 = jnp.where(b2, col(5), col(4))
        u0 = jnp.where(b3, t1, t0)
        u1 = jnp.where(b3, col(6), t2)
        pv = jnp.where(b4, u1, u0)
        cv = jnp.where(b1, col(10), jnp.where(b0, col(9), col(8)))
        out_ref[si] = pv + cv

    def xlu_square(si):
        tbl21 = tbl21_ref[si]
        pc = p_ref[si][None, :] * 3 + c_ref[si][None, :]
        idxb = jnp.broadcast_to(pc, (D, BLKB2))
        out_ref[si] = jnp.take_along_axis(tbl21, idxb, axis=1)

    def group(g, carry):
        base = g * 4
        xlu_square(base)
        valu_square(base + 1)
        valu_square(base + 2)
        valu_square(base + 3)
        return carry

    lax.fori_loop(0, 2, group, 0, unroll=1)


def _tc_lookup(tbl, tbl21, pT, cT):
    B = pT.shape[1]
    return pl.pallas_call(
        _tc_body,
        grid=(S // 8, B // BLKB2),
        in_specs=[pl.BlockSpec((8, D, 16), lambda s, j: (s, 0, 0)),
                  pl.BlockSpec((8, D, 32), lambda s, j: (s, 0, 0)),
                  pl.BlockSpec((8, BLKB2), lambda s, j: (s, j)),
                  pl.BlockSpec((8, BLKB2), lambda s, j: (s, j))],
        out_specs=pl.BlockSpec((8, D, BLKB2), lambda s, j: (s, 0, j)),
        out_shape=jax.ShapeDtypeStruct((S, D, B), jnp.float32),
    )(tbl, tbl21, pT, cT)


def kernel(pieces_ids, color_ids, position_table, piece_table, color_table,
           indexes):
    B, _ = pieces_ids.shape
    pos = jnp.take(position_table, indexes, axis=0)
    pcols = piece_table[jnp.clip(jnp.arange(8), 0, 6)].T
    ccols = color_table[jnp.clip(jnp.arange(8), 0, 2)].T
    ptab = pos[:, :, None] + pcols[None, :, :]
    ctab = jnp.broadcast_to(ccols[None, :, :], (S, D, 8))
    tbl = jnp.concatenate([ptab, ctab], axis=-1)
    pc32 = jnp.arange(32)
    p21 = piece_table[jnp.clip(pc32 // 3, 0, 6)].T
    c21 = color_table[pc32 % 3].T
    tbl21 = (pos[:, :, None] + p21[None, :, :]) + c21[None, :, :]
    pT = pieces_ids.astype(jnp.int32).T
    cT = color_ids.astype(jnp.int32).T
    out3 = _tc_lookup(tbl, tbl21, pT, cT)
    return jnp.transpose(out3, (2, 0, 1))
